# baseline (speedup 1.0000x reference)
import functools

import jax
import jax.numpy as jnp
from jax import lax
from jax.experimental import pallas as pl
from jax.experimental.pallas import tpu as pltpu
from jax.experimental.pallas import tpu_sc as plsc

N = 10000
E = 320000
NC = 2
NS = 16
NW = NC * NS
CHUNK = 128
N_CHUNKS = E // CHUNK
NBUF = 4
CPB_DEG = 80
CPB = 160
EPC = NS * CPB + NBUF
NP = 10112
ROWS_PER_SUB = NP // NS
SINK = N
RB = 2000

_MESH = plsc.VectorSubcoreMesh(core_axis_name="c", subcore_axis_name="s")
_SC_PARAMS = pltpu.CompilerParams(use_tc_tiling_on_sc=False)


def _sc_degree(ei2, ones, zeros16):
  @functools.partial(
      pl.kernel,
      out_type=jax.ShapeDtypeStruct((NC, NP, 16), jnp.float32),
      mesh=_MESH,
      compiler_params=_SC_PARAMS,
      scratch_types=[
          pltpu.VMEM((CPB_DEG, 2, CHUNK), jnp.int32),
          pltpu.VMEM((CHUNK, 16), jnp.float32),
          pltpu.VMEM_SHARED((NP, 16), jnp.float32),
      ],
  )
  def k(dst_hbm, ones_hbm, z_hbm, out_hbm, idx_d, ones_v, acc_sh):
    c = lax.axis_index("c")
    s = lax.axis_index("s")
    wid = s * NC + c
    rb = s * ROWS_PER_SUB
    pltpu.sync_copy(z_hbm.at[pl.ds(rb, ROWS_PER_SUB)],
                    acc_sh.at[pl.ds(rb, ROWS_PER_SUB)])
    pltpu.sync_copy(dst_hbm.at[pl.ds(wid * CPB_DEG, CPB_DEG)], idx_d)
    pltpu.sync_copy(ones_hbm, ones_v)
    plsc.subcore_barrier()

    def body(j, carry):
      pltpu.sync_copy(ones_v, acc_sh.at[idx_d.at[j, jnp.int32(1)]], add=True)
      return carry

    lax.fori_loop(jnp.int32(0), jnp.int32(CPB_DEG), body, jnp.int32(0))
    plsc.subcore_barrier()
    pltpu.sync_copy(acc_sh.at[pl.ds(rb, ROWS_PER_SUB)],
                    out_hbm.at[c, pl.ds(rb, ROWS_PER_SUB)])

  return k(ei2, ones, zeros16)


def _sc_scatter(ya, yb, ei2, zeros, half):
  @functools.partial(
      pl.kernel,
      out_type=jax.ShapeDtypeStruct((NC, NP, half), jnp.float32),
      mesh=_MESH,
      compiler_params=_SC_PARAMS,
      scratch_types=[
          pltpu.VMEM((4, 2, CHUNK), jnp.int32),
          pltpu.VMEM((2, CHUNK, half), jnp.float32),
          pltpu.VMEM_SHARED((NP, half), jnp.float32),
          pltpu.VMEM_SHARED((NP, half), jnp.float32),
      ] + [pltpu.SemaphoreType.DMA] * 6,
  )
  def k(ya_hbm, yb_hbm, ei_hbm, z_hbm, out_hbm, idx, rows, y_sh, acc_sh,
        *sems):
    c = lax.axis_index("c")
    s = lax.axis_index("s")
    rb = s * ROWS_PER_SUB
    base = s * CPB
    pltpu.sync_copy(z_hbm.at[pl.ds(rb, ROWS_PER_SUB)],
                    acc_sh.at[pl.ds(rb, ROWS_PER_SUB)])

    def stage(y_hbm):
      @pl.when(s < NS - 1)
      def _():
        pltpu.sync_copy(y_hbm.at[pl.ds(rb, ROWS_PER_SUB)],
                        y_sh.at[pl.ds(rb, ROWS_PER_SUB)])

      @pl.when(s == NS - 1)
      def _():
        last = N - (NS - 1) * ROWS_PER_SUB
        pltpu.sync_copy(y_hbm.at[pl.ds(rb, last)],
                        y_sh.at[pl.ds(rb, last)])

    @pl.when(c == 0)
    def _():
      stage(ya_hbm)

    @pl.when(c == 1)
    def _():
      stage(yb_hbm)

    plsc.subcore_barrier()

    isem, gsem = sems[:4], sems[4:]

    def load_idx(j, slot):
      pltpu.async_copy(ei_hbm.at[j], idx.at[jnp.int32(slot)], isem[int(slot)])

    def wait_idx(slot):
      pltpu.make_async_copy(ei_hbm.at[jnp.int32(0)], idx.at[jnp.int32(slot)],
                            isem[int(slot)]).wait()

    def gather(islot, rslot):
      pltpu.async_copy(y_sh.at[idx.at[jnp.int32(islot), jnp.int32(0)]],
                       rows.at[jnp.int32(rslot)], gsem[int(rslot)])

    def wait_gather(rslot):
      pltpu.make_async_copy(y_sh.at[idx.at[jnp.int32(0), jnp.int32(0)]],
                            rows.at[jnp.int32(rslot)],
                            gsem[int(rslot)]).wait()

    load_idx(base + jnp.int32(0), 0)
    load_idx(base + jnp.int32(1), 1)

    def group(g, carry):
      j0 = base + g * jnp.int32(4)
      for u in (0, 2):
        wait_idx(u)
        wait_idx(u + 1)
        gather(u, 0)
        gather(u + 1, 1)
        wait_gather(0)
        pltpu.sync_copy(rows.at[jnp.int32(0)],
                        acc_sh.at[idx.at[jnp.int32(u), jnp.int32(1)]],
                        add=True)
        wait_gather(1)
        pltpu.sync_copy(rows.at[jnp.int32(1)],
                        acc_sh.at[idx.at[jnp.int32(u + 1), jnp.int32(1)]],
                        add=True)
        load_idx(j0 + jnp.int32(u + 2), (u + 2) % 4)
        load_idx(j0 + jnp.int32(u + 3), (u + 3) % 4)
      return carry

    lax.fori_loop(jnp.int32(0), jnp.int32(CPB // 4), group, jnp.int32(0))
    wait_idx(0)
    wait_idx(1)

    plsc.subcore_barrier()
    pltpu.sync_copy(acc_sh.at[pl.ds(rb, ROWS_PER_SUB)],
                    out_hbm.at[c, pl.ds(rb, ROWS_PER_SUB)])

  return k(ya, yb, ei2, zeros)


def _dinv(da_ref, db_ref):
  deg = 1.0 + da_ref[:, :1] + db_ref[:, :1]
  return lax.rsqrt(deg)


def _tc_prescale(x, W, dega, degb):
  half = W.shape[1] // 2

  def body(x_ref, w_ref, da_ref, db_ref, ya_ref, yb_ref):
    xw = jnp.dot(x_ref[...], w_ref[...], preferred_element_type=jnp.float32,
                 precision=lax.Precision.HIGHEST)
    y = xw * _dinv(da_ref, db_ref)
    ya_ref[...] = y[:, :half]
    yb_ref[...] = y[:, half:]

  return pl.pallas_call(
      body,
      grid=(N // RB,),
      in_specs=[
          pl.BlockSpec((RB, x.shape[1]), lambda i: (i, jnp.int32(0))),
          pl.BlockSpec(W.shape, lambda i: (jnp.int32(0), jnp.int32(0))),
          pl.BlockSpec((RB, 16), lambda i: (i, jnp.int32(0))),
          pl.BlockSpec((RB, 16), lambda i: (i, jnp.int32(0))),
      ],
      out_specs=(pl.BlockSpec((RB, half), lambda i: (i, jnp.int32(0))),
                 pl.BlockSpec((RB, half), lambda i: (i, jnp.int32(0)))),
      out_shape=(jax.ShapeDtypeStruct((N, half), jnp.float32),
                 jax.ShapeDtypeStruct((N, half), jnp.float32)),
  )(x, W, dega, degb)


def _tc_mid(ya, yb, a0, a1, dega, degb, b1, W2):
  half = W2.shape[1] // 2

  def body(ya_ref, yb_ref, a0_ref, a1_ref, da_ref, db_ref, b_ref, w_ref,
           oa_ref, ob_ref):
    dinv = _dinv(da_ref, db_ref)
    y = jnp.concatenate([ya_ref[...], yb_ref[...]], axis=1)
    a = jnp.concatenate([a0_ref[...], a1_ref[...]], axis=1)
    h = jnp.maximum((y + a) * dinv + b_ref[...], 0.0)
    hw = jnp.dot(h, w_ref[...], preferred_element_type=jnp.float32,
                 precision=lax.Precision.HIGHEST)
    y2 = hw * dinv
    oa_ref[...] = y2[:, :half]
    ob_ref[...] = y2[:, half:]

  h0 = ya.shape[1]
  return pl.pallas_call(
      body,
      grid=(N // RB,),
      in_specs=[
          pl.BlockSpec((RB, h0), lambda i: (i, jnp.int32(0))),
          pl.BlockSpec((RB, h0), lambda i: (i, jnp.int32(0))),
          pl.BlockSpec((RB, h0), lambda i: (i, jnp.int32(0))),
          pl.BlockSpec((RB, h0), lambda i: (i, jnp.int32(0))),
          pl.BlockSpec((RB, 16), lambda i: (i, jnp.int32(0))),
          pl.BlockSpec((RB, 16), lambda i: (i, jnp.int32(0))),
          pl.BlockSpec(b1.shape, lambda i: (jnp.int32(0), jnp.int32(0))),
          pl.BlockSpec(W2.shape, lambda i: (jnp.int32(0), jnp.int32(0))),
      ],
      out_specs=(pl.BlockSpec((RB, half), lambda i: (i, jnp.int32(0))),
                 pl.BlockSpec((RB, half), lambda i: (i, jnp.int32(0)))),
      out_shape=(jax.ShapeDtypeStruct((N, half), jnp.float32),
                 jax.ShapeDtypeStruct((N, half), jnp.float32)),
  )(ya, yb, a0, a1, dega, degb, b1, W2)


def _tc_final(ya, yb, a0, a1, dega, degb, b2):
  def body(ya_ref, yb_ref, a0_ref, a1_ref, da_ref, db_ref, b_ref, o_ref):
    dinv = _dinv(da_ref, db_ref)
    y = jnp.concatenate([ya_ref[...], yb_ref[...]], axis=1)
    a = jnp.concatenate([a0_ref[...], a1_ref[...]], axis=1)
    o_ref[...] = (y + a) * dinv + b_ref[...]

  h0 = ya.shape[1]
  return pl.pallas_call(
      body,
      grid=(N // RB,),
      in_specs=[
          pl.BlockSpec((RB, h0), lambda i: (i, jnp.int32(0))),
          pl.BlockSpec((RB, h0), lambda i: (i, jnp.int32(0))),
          pl.BlockSpec((RB, h0), lambda i: (i, jnp.int32(0))),
          pl.BlockSpec((RB, h0), lambda i: (i, jnp.int32(0))),
          pl.BlockSpec((RB, 16), lambda i: (i, jnp.int32(0))),
          pl.BlockSpec((RB, 16), lambda i: (i, jnp.int32(0))),
          pl.BlockSpec(b2.shape, lambda i: (jnp.int32(0), jnp.int32(0))),
      ],
      out_specs=pl.BlockSpec((RB, 2 * h0), lambda i: (i, jnp.int32(0))),
      out_shape=jax.ShapeDtypeStruct((N, 2 * ya.shape[1]), jnp.float32),
  )(ya, yb, a0, a1, dega, degb, b2)


def kernel(x, ei, W1, b1, W_mean, b_mean):
  x = x.astype(jnp.float32)
  W1 = W1.astype(jnp.float32)
  W_mean = W_mean.astype(jnp.float32)
  b1 = b1.astype(jnp.float32)
  b_mean = b_mean.astype(jnp.float32)
  src = ei[0].astype(jnp.int32)
  dst = ei[1].astype(jnp.int32)
  d_hid = W1.shape[1]
  d_emb = W_mean.shape[1]

  pad = EPC * CHUNK - E
  src2 = jnp.concatenate([src, jnp.zeros((pad,), jnp.int32)]).reshape(EPC, CHUNK)
  dst2 = jnp.concatenate([dst, jnp.full((pad,), SINK, jnp.int32)]).reshape(EPC, CHUNK)
  ei2 = jnp.stack([src2, dst2], axis=1)

  zeros16 = jnp.zeros((NP, 16), jnp.float32)
  zeros_h = jnp.zeros((NP, d_hid // 2), jnp.float32)
  zeros_e = jnp.zeros((NP, d_emb // 2), jnp.float32)
  ones = jnp.ones((CHUNK, 16), jnp.float32)

  degp = _sc_degree(ei2, ones, zeros16)
  dega, degb = degp[0, :N], degp[1, :N]

  y1a, y1b = _tc_prescale(x, W1, dega, degb)
  acc1 = _sc_scatter(y1a, y1b, ei2, zeros_h, d_hid // 2)
  y2a, y2b = _tc_mid(y1a, y1b, acc1[0, :N], acc1[1, :N], dega, degb,
                     b1.reshape(1, -1), W_mean)
  acc2 = _sc_scatter(y2a, y2b, ei2, zeros_e, d_emb // 2)
  mean = _tc_final(y2a, y2b, acc2[0, :N], acc2[1, :N], dega, degb,
                   b_mean.reshape(1, -1))

  return (mean, jnp.zeros((1,), jnp.float32))


# speedup vs baseline: 1.0416x; 1.0416x over previous
import functools

import jax
import jax.numpy as jnp
from jax import lax
from jax.experimental import pallas as pl
from jax.experimental.pallas import tpu as pltpu
from jax.experimental.pallas import tpu_sc as plsc

N = 10000
E = 320000
NC = 2
NS = 16
NW = NC * NS
CHUNK = 128
N_CHUNKS = E // CHUNK
NBUF = 4
CPB_DEG = 80
CPB = 160
EPC = NS * CPB + NBUF
NP = 10112
ROWS_PER_SUB = NP // NS
SINK = N
RB = 2000

_MESH = plsc.VectorSubcoreMesh(core_axis_name="c", subcore_axis_name="s")
_SC_PARAMS = pltpu.CompilerParams(use_tc_tiling_on_sc=False)


def _sc_degree(ei2, ones, zeros16):
  @functools.partial(
      pl.kernel,
      out_type=jax.ShapeDtypeStruct((NC, NP, 16), jnp.float32),
      mesh=_MESH,
      compiler_params=_SC_PARAMS,
      scratch_types=[
          pltpu.VMEM((CPB_DEG, 2, CHUNK), jnp.int32),
          pltpu.VMEM((CHUNK, 16), jnp.float32),
          pltpu.VMEM_SHARED((NP, 16), jnp.float32),
      ],
  )
  def k(dst_hbm, ones_hbm, z_hbm, out_hbm, idx_d, ones_v, acc_sh):
    c = lax.axis_index("c")
    s = lax.axis_index("s")
    wid = s * NC + c
    rb = s * ROWS_PER_SUB
    pltpu.sync_copy(z_hbm.at[pl.ds(rb, ROWS_PER_SUB)],
                    acc_sh.at[pl.ds(rb, ROWS_PER_SUB)])
    pltpu.sync_copy(dst_hbm.at[pl.ds(wid * CPB_DEG, CPB_DEG)], idx_d)
    pltpu.sync_copy(ones_hbm, ones_v)
    plsc.subcore_barrier()

    def body(j, carry):
      pltpu.sync_copy(ones_v, acc_sh.at[idx_d.at[j, jnp.int32(1)]], add=True)
      return carry

    lax.fori_loop(jnp.int32(0), jnp.int32(CPB_DEG), body, jnp.int32(0))
    plsc.subcore_barrier()
    pltpu.sync_copy(acc_sh.at[pl.ds(rb, ROWS_PER_SUB)],
                    out_hbm.at[c, pl.ds(rb, ROWS_PER_SUB)])

  return k(ei2, ones, zeros16)


def _sc_scatter(ya, yb, ei2, zeros, half):
  @functools.partial(
      pl.kernel,
      out_type=jax.ShapeDtypeStruct((NC, NP, half), jnp.float32),
      mesh=_MESH,
      compiler_params=_SC_PARAMS,
      scratch_types=[
          pltpu.VMEM((4, 2, CHUNK), jnp.int32),
          pltpu.VMEM((CHUNK, half), jnp.float32),
          pltpu.VMEM_SHARED((NP, half), jnp.float32),
          pltpu.VMEM_SHARED((NP, half), jnp.float32),
      ] + [pltpu.SemaphoreType.DMA] * 4,
  )
  def k(ya_hbm, yb_hbm, ei_hbm, z_hbm, out_hbm, idx, rowbuf, y_sh, acc_sh,
        *isem):
    c = lax.axis_index("c")
    s = lax.axis_index("s")
    rb = s * ROWS_PER_SUB
    base = s * CPB
    pltpu.sync_copy(z_hbm.at[pl.ds(rb, ROWS_PER_SUB)],
                    acc_sh.at[pl.ds(rb, ROWS_PER_SUB)])

    def stage(y_hbm):
      @pl.when(s < NS - 1)
      def _():
        pltpu.sync_copy(y_hbm.at[pl.ds(rb, ROWS_PER_SUB)],
                        y_sh.at[pl.ds(rb, ROWS_PER_SUB)])

      @pl.when(s == NS - 1)
      def _():
        last = N - (NS - 1) * ROWS_PER_SUB
        pltpu.sync_copy(y_hbm.at[pl.ds(rb, last)],
                        y_sh.at[pl.ds(rb, last)])

    @pl.when(c == 0)
    def _():
      stage(ya_hbm)

    @pl.when(c == 1)
    def _():
      stage(yb_hbm)

    plsc.subcore_barrier()

    def load_idx(j, slot):
      pltpu.async_copy(ei_hbm.at[j], idx.at[jnp.int32(slot)], isem[int(slot)])

    def wait_idx(slot):
      pltpu.make_async_copy(ei_hbm.at[jnp.int32(0)], idx.at[jnp.int32(slot)],
                            isem[int(slot)]).wait()

    load_idx(base + jnp.int32(0), 0)
    load_idx(base + jnp.int32(1), 1)

    def group(g, carry):
      j0 = base + g * jnp.int32(4)
      for u in range(4):
        ui = jnp.int32(u)
        wait_idx(u)
        pltpu.sync_copy(y_sh.at[idx.at[ui, jnp.int32(0)]], rowbuf)
        pltpu.sync_copy(rowbuf, acc_sh.at[idx.at[ui, jnp.int32(1)]], add=True)
        load_idx(j0 + jnp.int32(u + 2), (u + 2) % 4)
      return carry

    lax.fori_loop(jnp.int32(0), jnp.int32(CPB // 4), group, jnp.int32(0))
    wait_idx(0)
    wait_idx(1)

    plsc.subcore_barrier()
    pltpu.sync_copy(acc_sh.at[pl.ds(rb, ROWS_PER_SUB)],
                    out_hbm.at[c, pl.ds(rb, ROWS_PER_SUB)])

  return k(ya, yb, ei2, zeros)


def _dinv(da_ref, db_ref):
  deg = 1.0 + da_ref[:, :1] + db_ref[:, :1]
  return lax.rsqrt(deg)


def _tc_prescale(x, W, dega, degb):
  half = W.shape[1] // 2

  def body(x_ref, w_ref, da_ref, db_ref, ya_ref, yb_ref):
    xw = jnp.dot(x_ref[...], w_ref[...], preferred_element_type=jnp.float32,
                 precision=lax.Precision.HIGHEST)
    y = xw * _dinv(da_ref, db_ref)
    ya_ref[...] = y[:, :half]
    yb_ref[...] = y[:, half:]

  return pl.pallas_call(
      body,
      grid=(N // RB,),
      in_specs=[
          pl.BlockSpec((RB, x.shape[1]), lambda i: (i, jnp.int32(0))),
          pl.BlockSpec(W.shape, lambda i: (jnp.int32(0), jnp.int32(0))),
          pl.BlockSpec((RB, 16), lambda i: (i, jnp.int32(0))),
          pl.BlockSpec((RB, 16), lambda i: (i, jnp.int32(0))),
      ],
      out_specs=(pl.BlockSpec((RB, half), lambda i: (i, jnp.int32(0))),
                 pl.BlockSpec((RB, half), lambda i: (i, jnp.int32(0)))),
      out_shape=(jax.ShapeDtypeStruct((N, half), jnp.float32),
                 jax.ShapeDtypeStruct((N, half), jnp.float32)),
  )(x, W, dega, degb)


def _tc_mid(ya, yb, a0, a1, dega, degb, b1, W2):
  half = W2.shape[1] // 2

  def body(ya_ref, yb_ref, a0_ref, a1_ref, da_ref, db_ref, b_ref, w_ref,
           oa_ref, ob_ref):
    dinv = _dinv(da_ref, db_ref)
    y = jnp.concatenate([ya_ref[...], yb_ref[...]], axis=1)
    a = jnp.concatenate([a0_ref[...], a1_ref[...]], axis=1)
    h = jnp.maximum((y + a) * dinv + b_ref[...], 0.0)
    hw = jnp.dot(h, w_ref[...], preferred_element_type=jnp.float32,
                 precision=lax.Precision.HIGHEST)
    y2 = hw * dinv
    oa_ref[...] = y2[:, :half]
    ob_ref[...] = y2[:, half:]

  h0 = ya.shape[1]
  return pl.pallas_call(
      body,
      grid=(N // RB,),
      in_specs=[
          pl.BlockSpec((RB, h0), lambda i: (i, jnp.int32(0))),
          pl.BlockSpec((RB, h0), lambda i: (i, jnp.int32(0))),
          pl.BlockSpec((RB, h0), lambda i: (i, jnp.int32(0))),
          pl.BlockSpec((RB, h0), lambda i: (i, jnp.int32(0))),
          pl.BlockSpec((RB, 16), lambda i: (i, jnp.int32(0))),
          pl.BlockSpec((RB, 16), lambda i: (i, jnp.int32(0))),
          pl.BlockSpec(b1.shape, lambda i: (jnp.int32(0), jnp.int32(0))),
          pl.BlockSpec(W2.shape, lambda i: (jnp.int32(0), jnp.int32(0))),
      ],
      out_specs=(pl.BlockSpec((RB, half), lambda i: (i, jnp.int32(0))),
                 pl.BlockSpec((RB, half), lambda i: (i, jnp.int32(0)))),
      out_shape=(jax.ShapeDtypeStruct((N, half), jnp.float32),
                 jax.ShapeDtypeStruct((N, half), jnp.float32)),
  )(ya, yb, a0, a1, dega, degb, b1, W2)


def _tc_final(ya, yb, a0, a1, dega, degb, b2):
  def body(ya_ref, yb_ref, a0_ref, a1_ref, da_ref, db_ref, b_ref, o_ref):
    dinv = _dinv(da_ref, db_ref)
    y = jnp.concatenate([ya_ref[...], yb_ref[...]], axis=1)
    a = jnp.concatenate([a0_ref[...], a1_ref[...]], axis=1)
    o_ref[...] = (y + a) * dinv + b_ref[...]

  h0 = ya.shape[1]
  return pl.pallas_call(
      body,
      grid=(N // RB,),
      in_specs=[
          pl.BlockSpec((RB, h0), lambda i: (i, jnp.int32(0))),
          pl.BlockSpec((RB, h0), lambda i: (i, jnp.int32(0))),
          pl.BlockSpec((RB, h0), lambda i: (i, jnp.int32(0))),
          pl.BlockSpec((RB, h0), lambda i: (i, jnp.int32(0))),
          pl.BlockSpec((RB, 16), lambda i: (i, jnp.int32(0))),
          pl.BlockSpec((RB, 16), lambda i: (i, jnp.int32(0))),
          pl.BlockSpec(b2.shape, lambda i: (jnp.int32(0), jnp.int32(0))),
      ],
      out_specs=pl.BlockSpec((RB, 2 * h0), lambda i: (i, jnp.int32(0))),
      out_shape=jax.ShapeDtypeStruct((N, 2 * ya.shape[1]), jnp.float32),
  )(ya, yb, a0, a1, dega, degb, b2)


def kernel(x, ei, W1, b1, W_mean, b_mean):
  x = x.astype(jnp.float32)
  W1 = W1.astype(jnp.float32)
  W_mean = W_mean.astype(jnp.float32)
  b1 = b1.astype(jnp.float32)
  b_mean = b_mean.astype(jnp.float32)
  src = ei[0].astype(jnp.int32)
  dst = ei[1].astype(jnp.int32)
  d_hid = W1.shape[1]
  d_emb = W_mean.shape[1]

  pad = EPC * CHUNK - E
  src2 = jnp.concatenate([src, jnp.zeros((pad,), jnp.int32)]).reshape(EPC, CHUNK)
  dst2 = jnp.concatenate([dst, jnp.full((pad,), SINK, jnp.int32)]).reshape(EPC, CHUNK)
  ei2 = jnp.stack([src2, dst2], axis=1)

  zeros16 = jnp.zeros((NP, 16), jnp.float32)
  zeros_h = jnp.zeros((NP, d_hid // 2), jnp.float32)
  zeros_e = jnp.zeros((NP, d_emb // 2), jnp.float32)
  ones = jnp.ones((CHUNK, 16), jnp.float32)

  degp = _sc_degree(ei2, ones, zeros16)
  dega, degb = degp[0, :N], degp[1, :N]

  y1a, y1b = _tc_prescale(x, W1, dega, degb)
  acc1 = _sc_scatter(y1a, y1b, ei2, zeros_h, d_hid // 2)
  y2a, y2b = _tc_mid(y1a, y1b, acc1[0, :N], acc1[1, :N], dega, degb,
                     b1.reshape(1, -1), W_mean)
  acc2 = _sc_scatter(y2a, y2b, ei2, zeros_e, d_emb // 2)
  mean = _tc_final(y2a, y2b, acc2[0, :N], acc2[1, :N], dega, degb,
                   b_mean.reshape(1, -1))

  return (mean, jnp.zeros((1,), jnp.float32))
